# baseline (speedup 1.0000x reference)
import jax
import jax.numpy as jnp
from jax.experimental import pallas as pl
from jax.experimental.pallas import tpu as pltpu


def _dotbf(a, b):
    return jax.lax.dot_general(
        a.astype(jnp.bfloat16), b.astype(jnp.bfloat16), (((1,), (0,)), ((), ())),
        preferred_element_type=jnp.float32)


def _leaky(x):
    return jnp.where(x >= 0, x, 0.1 * x)


def _body(xt_ref, w1t, b1c, w2t, b2c, w3t, b3c, w4t, b4c,
          embd, embd_hi, embd_lo, embd_t,
          w5t, b5c, w6t, b6c, w7t, b7c, w8t, b8c,
          xrt_ref, zenc_ref, zdec_ref, zfe_ref, runmin_ref):
    i = pl.program_id(0)
    blk = xt_ref.shape[1]
    K, D = embd.shape

    ht = jnp.maximum(_dotbf(w1t[...], xt_ref[...]) + b1c[...], 0.0)
    ht = jnp.maximum(_dotbf(w2t[...], ht) + b2c[...], 0.0)
    ht = jnp.maximum(_dotbf(w3t[...], ht) + b3c[...], 0.0)
    zt = _dotbf(w4t[...], ht) + b4c[...]
    z = zt.T
    zenc_ref[...] = z

    qsq = jnp.sum(z * z, axis=1, keepdims=True)
    tsq = jnp.sum(embd[...] * embd[...], axis=1)
    g = _dotbf(z, embd_t[...])
    d2 = jnp.maximum(qsq + tsq[None, :] - 2.0 * g, 0.0)

    iota_k = jax.lax.broadcasted_iota(jnp.int32, (blk, K), 1)
    dmin = jnp.min(d2, axis=1, keepdims=True)
    idx = jnp.min(jnp.where(d2 == dmin, iota_k, K), axis=1, keepdims=True)
    onehot = (iota_k == idx).astype(jnp.bfloat16)
    zq = _dotbf(onehot, embd_hi[...]) + _dotbf(onehot, embd_lo[...])
    zdec_ref[...] = zq

    iota_r = jax.lax.broadcasted_iota(jnp.int32, (blk, K), 0)
    bmin = jnp.min(d2, axis=0)
    brow = jnp.min(jnp.where(d2 == bmin[None, :], iota_r, blk), axis=0)

    @pl.when(i == 0)
    def _():
        runmin_ref[...] = jnp.full(runmin_ref.shape, jnp.inf, jnp.float32)

    bmin_c = bmin.reshape(K, 1)
    brow_c = brow.reshape(K, 1)
    better = bmin_c < runmin_ref[...]
    runmin_ref[...] = jnp.where(better, bmin_c, runmin_ref[...])
    iota_b = jax.lax.broadcasted_iota(jnp.int32, (K, blk), 1)
    onehot2 = (iota_b == brow_c).astype(jnp.bfloat16)
    z_hi = z.astype(jnp.bfloat16)
    z_lo = (z - z_hi.astype(jnp.float32)).astype(jnp.bfloat16)
    rows = _dotbf(onehot2, z_hi) + _dotbf(onehot2, z_lo)
    zfe_ref[...] = jnp.where(better, rows, zfe_ref[...])

    dt = _leaky(_dotbf(w5t[...], zq.T) + b5c[...])
    dt = _leaky(_dotbf(w6t[...], dt) + b6c[...])
    dt = _leaky(_dotbf(w7t[...], dt) + b7c[...])
    xrt_ref[...] = jnp.tanh(_dotbf(w8t[...], dt) + b8c[...])


def kernel(X, W1, b1, W2, b2, W3, b3, W4, b4, embd, W5, b5, W6, b6, W7, b7, W8, b8):
    B = X.shape[0]
    K, D = embd.shape
    BLK = 512
    nblk = B // BLK

    XT = X.transpose(2, 3, 1, 0).reshape(784, B)

    def full(arr):
        return pl.BlockSpec(arr.shape, lambda i: (0,) * arr.ndim)

    row2 = lambda n: pl.BlockSpec((BLK, n), lambda i: (i, 0))
    colT = pl.BlockSpec((784, BLK), lambda i: (0, i))
    bf = lambda w: w.astype(jnp.bfloat16)
    embd_hi = bf(embd)
    embd_lo = bf(embd - embd_hi.astype(jnp.float32))
    col = lambda b: b.reshape(-1, 1)
    weights = (bf(W1.T), col(b1), bf(W2.T), col(b2), bf(W3.T), col(b3),
               bf(W4.T), col(b4), embd, embd_hi, embd_lo, bf(embd.T),
               bf(W5.T), col(b5), bf(W6.T), col(b6), bf(W7.T), col(b7),
               bf(W8.T), col(b8))

    out = pl.pallas_call(
        _body,
        grid=(nblk,),
        in_specs=[colT] + [full(w) for w in weights],
        out_specs=[colT, row2(D), row2(D),
                   pl.BlockSpec((K, D), lambda i: (0, 0))],
        out_shape=[
            jax.ShapeDtypeStruct((784, B), jnp.float32),
            jax.ShapeDtypeStruct((B, D), jnp.float32),
            jax.ShapeDtypeStruct((B, D), jnp.float32),
            jax.ShapeDtypeStruct((K, D), jnp.float32),
        ],
        scratch_shapes=[pltpu.VMEM((K, 1), jnp.float32)],
        compiler_params=pltpu.CompilerParams(
            dimension_semantics=("arbitrary",)),
    )(XT, *weights)

    XRT, Z_enc, Z_dec, Zfe = out
    return (XRT.T.reshape(B, 1, 28, 28), Z_enc, Z_dec, Zfe)


# speedup vs baseline: 1.6489x; 1.6489x over previous
import jax
import jax.numpy as jnp
from jax.experimental import pallas as pl
from jax.experimental.pallas import tpu as pltpu


def _dotbf(a, b):
    return jax.lax.dot_general(
        a.astype(jnp.bfloat16), b.astype(jnp.bfloat16), (((1,), (0,)), ((), ())),
        preferred_element_type=jnp.float32)


def _leaky(x):
    return jnp.where(x >= 0, x, 0.1 * x)


def _body(xt_ref, w1t, b1c, w2t, b2c, w3t, b3c, w4t, b4c,
          embd, embd_hi, embd_lo, embd_t,
          w5t, b5c, w6t, b6c, w7t, b7c, w8t, b8c,
          xrt_ref, zenc_ref, zdec_ref, zfe_ref, runmin_ref):
    i = pl.program_id(0)
    blk = xt_ref.shape[1]
    K, D = embd.shape

    ht = jnp.maximum(_dotbf(w1t[...], xt_ref[...]) + b1c[...], 0.0)
    ht = jnp.maximum(_dotbf(w2t[...], ht) + b2c[...], 0.0)
    ht = jnp.maximum(_dotbf(w3t[...], ht) + b3c[...], 0.0)
    zt = _dotbf(w4t[...], ht) + b4c[...]
    z = zt.T
    zenc_ref[...] = z

    qsq = jnp.sum(z * z, axis=1, keepdims=True)
    tsq = jnp.sum(embd[...] * embd[...], axis=1)
    g = _dotbf(z, embd_t[...])
    d2 = jnp.maximum(qsq + tsq[None, :] - 2.0 * g, 0.0)

    iota_k = jax.lax.broadcasted_iota(jnp.int32, (blk, K), 1)
    dmin = jnp.min(d2, axis=1, keepdims=True)
    idx = jnp.min(jnp.where(d2 == dmin, iota_k, K), axis=1, keepdims=True)
    onehot = (iota_k == idx).astype(jnp.bfloat16)
    zq = _dotbf(onehot, embd_hi[...]) + _dotbf(onehot, embd_lo[...])
    zdec_ref[...] = zq

    iota_r = jax.lax.broadcasted_iota(jnp.int32, (blk, K), 0)
    bmin = jnp.min(d2, axis=0)
    brow = jnp.min(jnp.where(d2 == bmin[None, :], iota_r, blk), axis=0)

    @pl.when(i == 0)
    def _():
        runmin_ref[...] = jnp.full(runmin_ref.shape, jnp.inf, jnp.float32)

    bmin_c = bmin.reshape(K, 1)
    brow_c = brow.reshape(K, 1)
    better = bmin_c < runmin_ref[...]
    runmin_ref[...] = jnp.where(better, bmin_c, runmin_ref[...])
    iota_b = jax.lax.broadcasted_iota(jnp.int32, (K, blk), 1)
    onehot2 = (iota_b == brow_c).astype(jnp.bfloat16)
    z_hi = z.astype(jnp.bfloat16)
    z_lo = (z - z_hi.astype(jnp.float32)).astype(jnp.bfloat16)
    rows = _dotbf(onehot2, z_hi) + _dotbf(onehot2, z_lo)
    zfe_ref[...] = jnp.where(better, rows, zfe_ref[...])

    dt = _leaky(_dotbf(w5t[...], zq.T) + b5c[...])
    dt = _leaky(_dotbf(w6t[...], dt) + b6c[...])
    dt = _leaky(_dotbf(w7t[...], dt) + b7c[...])
    xrt_ref[...] = jnp.tanh(_dotbf(w8t[...], dt) + b8c[...])


def kernel(X, W1, b1, W2, b2, W3, b3, W4, b4, embd, W5, b5, W6, b6, W7, b7, W8, b8):
    B = X.shape[0]
    K, D = embd.shape
    BLK = 512
    nblk = B // BLK

    XT = X[:, 0].transpose(1, 2, 0).reshape(784, B)

    def full(arr):
        return pl.BlockSpec(arr.shape, lambda i: (0,) * arr.ndim)

    row2 = lambda n: pl.BlockSpec((BLK, n), lambda i: (i, 0))
    colT = pl.BlockSpec((784, BLK), lambda i: (0, i))
    bf = lambda w: w.astype(jnp.bfloat16)
    embd_hi = bf(embd)
    embd_lo = bf(embd - embd_hi.astype(jnp.float32))
    col = lambda b: b.reshape(-1, 1)
    weights = (bf(W1.T), col(b1), bf(W2.T), col(b2), bf(W3.T), col(b3),
               bf(W4.T), col(b4), embd, embd_hi, embd_lo, bf(embd.T),
               bf(W5.T), col(b5), bf(W6.T), col(b6), bf(W7.T), col(b7),
               bf(W8.T), col(b8))

    out = pl.pallas_call(
        _body,
        grid=(nblk,),
        in_specs=[colT] + [full(w) for w in weights],
        out_specs=[colT, row2(D), row2(D),
                   pl.BlockSpec((K, D), lambda i: (0, 0))],
        out_shape=[
            jax.ShapeDtypeStruct((784, B), jnp.float32),
            jax.ShapeDtypeStruct((B, D), jnp.float32),
            jax.ShapeDtypeStruct((B, D), jnp.float32),
            jax.ShapeDtypeStruct((K, D), jnp.float32),
        ],
        scratch_shapes=[pltpu.VMEM((K, 1), jnp.float32)],
        compiler_params=pltpu.CompilerParams(
            dimension_semantics=("arbitrary",)),
    )(XT, *weights)

    XRT, Z_enc, Z_dec, Zfe = out
    return (XRT.T.reshape(B, 1, 28, 28), Z_enc, Z_dec, Zfe)


# speedup vs baseline: 1.8494x; 1.1216x over previous
import jax
import jax.numpy as jnp
from jax.experimental import pallas as pl
from jax.experimental.pallas import tpu as pltpu


def _dotbf(a, b):
    return jax.lax.dot_general(
        a.astype(jnp.bfloat16), b.astype(jnp.bfloat16), (((1,), (0,)), ((), ())),
        preferred_element_type=jnp.float32)


def _leaky(x):
    return jnp.where(x >= 0, x, 0.1 * x)


def _body(xt_ref, w1t, b1c, w2t, b2c, w3t, b3c, w4t, b4c,
          embd, embd_hi, embd_lo, embd_t,
          w5t, b5c, w6t, b6c, w7t, b7c, w8t, b8c,
          xrt_ref, zenc_ref, zdec_ref, zfe_ref, runmin_ref):
    i = pl.program_id(0)
    blk = xt_ref.shape[1]
    K, D = embd.shape

    ht = jnp.maximum(_dotbf(w1t[...], xt_ref[...]) + b1c[...], 0.0)
    ht = jnp.maximum(_dotbf(w2t[...], ht) + b2c[...], 0.0)
    ht = jnp.maximum(_dotbf(w3t[...], ht) + b3c[...], 0.0)
    zt = _dotbf(w4t[...], ht) + b4c[...]
    z = zt.T
    zenc_ref[...] = z

    qsq = jnp.sum(z * z, axis=1, keepdims=True)
    tsq = jnp.sum(embd[...] * embd[...], axis=1)
    g = _dotbf(z, embd_t[...])
    d2 = jnp.maximum(qsq + tsq[None, :] - 2.0 * g, 0.0)

    iota_k = jax.lax.broadcasted_iota(jnp.int32, (blk, K), 1)
    dmin = jnp.min(d2, axis=1, keepdims=True)
    idx = jnp.min(jnp.where(d2 == dmin, iota_k, K), axis=1, keepdims=True)
    onehot = (iota_k == idx).astype(jnp.bfloat16)
    zq = _dotbf(onehot, embd_hi[...]) + _dotbf(onehot, embd_lo[...])
    zdec_ref[...] = zq

    iota_r = jax.lax.broadcasted_iota(jnp.int32, (blk, K), 0)
    bmin = jnp.min(d2, axis=0)
    brow = jnp.min(jnp.where(d2 == bmin[None, :], iota_r, blk), axis=0)

    @pl.when(i == 0)
    def _():
        runmin_ref[...] = jnp.full(runmin_ref.shape, jnp.inf, jnp.float32)

    bmin_c = bmin.reshape(K, 1)
    brow_c = brow.reshape(K, 1)
    better = bmin_c < runmin_ref[...]
    runmin_ref[...] = jnp.where(better, bmin_c, runmin_ref[...])
    iota_b = jax.lax.broadcasted_iota(jnp.int32, (K, blk), 1)
    onehot2 = (iota_b == brow_c).astype(jnp.bfloat16)
    z_hi = z.astype(jnp.bfloat16)
    z_lo = (z - z_hi.astype(jnp.float32)).astype(jnp.bfloat16)
    rows = _dotbf(onehot2, z_hi) + _dotbf(onehot2, z_lo)
    zfe_ref[...] = jnp.where(better, rows, zfe_ref[...])

    dt = _leaky(_dotbf(w5t[...], zq.T) + b5c[...])
    dt = _leaky(_dotbf(w6t[...], dt) + b6c[...])
    dt = _leaky(_dotbf(w7t[...], dt) + b7c[...])
    xrt_ref[...] = jnp.tanh(_dotbf(w8t[...], dt) + b8c[...])


def kernel(X, W1, b1, W2, b2, W3, b3, W4, b4, embd, W5, b5, W6, b6, W7, b7, W8, b8):
    B = X.shape[0]
    K, D = embd.shape
    BLK = 512
    nblk = B // BLK

    XT = X[:, 0].transpose(1, 2, 0).reshape(784, B)

    def full(arr):
        return pl.BlockSpec(arr.shape, lambda i: (0,) * arr.ndim)

    row2 = lambda n: pl.BlockSpec((BLK, n), lambda i: (i, 0))
    colT = pl.BlockSpec((784, BLK), lambda i: (0, i))
    bf = lambda w: w.astype(jnp.bfloat16)
    embd_hi = bf(embd)
    embd_lo = bf(embd - embd_hi.astype(jnp.float32))
    col = lambda b: b.reshape(-1, 1)
    weights = (bf(W1.T), col(b1), bf(W2.T), col(b2), bf(W3.T), col(b3),
               bf(W4.T), col(b4), embd, embd_hi, embd_lo, bf(embd.T),
               bf(W5.T), col(b5), bf(W6.T), col(b6), bf(W7.T), col(b7),
               bf(W8.T), col(b8))

    out = pl.pallas_call(
        _body,
        grid=(nblk,),
        in_specs=[colT] + [full(w) for w in weights],
        out_specs=[colT, row2(D), row2(D),
                   pl.BlockSpec((K, D), lambda i: (0, 0))],
        out_shape=[
            jax.ShapeDtypeStruct((784, B), jnp.float32),
            jax.ShapeDtypeStruct((B, D), jnp.float32),
            jax.ShapeDtypeStruct((B, D), jnp.float32),
            jax.ShapeDtypeStruct((K, D), jnp.float32),
        ],
        scratch_shapes=[pltpu.VMEM((K, 1), jnp.float32)],
        compiler_params=pltpu.CompilerParams(
            dimension_semantics=("arbitrary",)),
    )(XT, *weights)

    XRT, Z_enc, Z_dec, Zfe = out
    X_recon = XRT.reshape(28, 28, B).transpose(2, 0, 1)[:, None, :, :]
    return (X_recon, Z_enc, Z_dec, Zfe)


# speedup vs baseline: 1.9328x; 1.0451x over previous
import jax
import jax.numpy as jnp
from jax.experimental import pallas as pl
from jax.experimental.pallas import tpu as pltpu
from jax.experimental.pallas import tpu_sc as plsc


def _dotbf(a, b):
    return jax.lax.dot_general(
        a.astype(jnp.bfloat16), b.astype(jnp.bfloat16), (((1,), (0,)), ((), ())),
        preferred_element_type=jnp.float32)


def _leaky(x):
    return jnp.where(x >= 0, x, 0.1 * x)


def _sc_gather(table, idx, n, d):
    @pl.kernel(out_type=jax.ShapeDtypeStruct((n, d), table.dtype),
               mesh=plsc.VectorSubcoreMesh(
                   core_axis_name="core", subcore_axis_name="subcore"))
    def k(tab_hbm, i_hbm, o_hbm):
        def body(i_vmem, o_vmem):
            pltpu.sync_copy(tab_hbm.at[i_vmem.at[0]], o_vmem)

        pltpu.emit_pipeline(
            body,
            grid=(n // 128,),
            in_specs=[pl.BlockSpec((1, 128), lambda i: (0, i))],
            out_specs=[pl.BlockSpec((128, d), lambda i: (i, 0))],
            core_axis_name=("core", "subcore"),
            dimension_semantics=(pltpu.PARALLEL,),
        )(i_hbm, o_hbm)

    return k(table, idx)


def _body(xt_ref, w1t, b1c, w2t, b2c, w3t, b3c, w4t, b4c,
          embd, embd_hi, embd_lo, embd_t,
          w5t, b5c, w6t, b6c, w7t, b7c, w8t, b8c,
          xrt_ref, zenc_ref, zdec_ref, zfi_ref, runmin_ref):
    i = pl.program_id(0)
    blk = xt_ref.shape[1]
    K, D = embd.shape

    ht = jnp.maximum(_dotbf(w1t[...], xt_ref[...]) + b1c[...], 0.0)
    ht = jnp.maximum(_dotbf(w2t[...], ht) + b2c[...], 0.0)
    ht = jnp.maximum(_dotbf(w3t[...], ht) + b3c[...], 0.0)
    zt = _dotbf(w4t[...], ht) + b4c[...]
    z = zt.T
    zenc_ref[...] = z

    qsq = jnp.sum(z * z, axis=1, keepdims=True)
    tsq = jnp.sum(embd[...] * embd[...], axis=1)
    g = _dotbf(z, embd_t[...])
    d2 = jnp.maximum(qsq + tsq[None, :] - 2.0 * g, 0.0)

    iota_k = jax.lax.broadcasted_iota(jnp.int32, (blk, K), 1)
    dmin = jnp.min(d2, axis=1, keepdims=True)
    idx = jnp.min(jnp.where(d2 == dmin, iota_k, K), axis=1, keepdims=True)
    onehot = (iota_k == idx).astype(jnp.bfloat16)
    zq = _dotbf(onehot, embd_hi[...]) + _dotbf(onehot, embd_lo[...])
    zdec_ref[...] = zq

    iota_r = jax.lax.broadcasted_iota(jnp.int32, (blk, K), 0)
    bmin = jnp.min(d2, axis=0).reshape(1, K)
    brow = jnp.min(jnp.where(d2 == bmin, iota_r, blk), axis=0).reshape(1, K)

    @pl.when(i == 0)
    def _():
        runmin_ref[...] = jnp.full(runmin_ref.shape, jnp.inf, jnp.float32)

    better = bmin < runmin_ref[...]
    runmin_ref[...] = jnp.where(better, bmin, runmin_ref[...])
    zfi_ref[...] = jnp.where(better, i * blk + brow, zfi_ref[...])

    dt = _leaky(_dotbf(w5t[...], zq.T) + b5c[...])
    dt = _leaky(_dotbf(w6t[...], dt) + b6c[...])
    dt = _leaky(_dotbf(w7t[...], dt) + b7c[...])
    xrt_ref[...] = jnp.tanh(_dotbf(w8t[...], dt) + b8c[...])


def kernel(X, W1, b1, W2, b2, W3, b3, W4, b4, embd, W5, b5, W6, b6, W7, b7, W8, b8):
    B = X.shape[0]
    K, D = embd.shape
    BLK = 512
    nblk = B // BLK

    XT = X[:, 0].transpose(1, 2, 0).reshape(784, B)

    def full(arr):
        return pl.BlockSpec(arr.shape, lambda i: (0,) * arr.ndim)

    row2 = lambda n: pl.BlockSpec((BLK, n), lambda i: (i, 0))
    colT = pl.BlockSpec((784, BLK), lambda i: (0, i))
    bf = lambda w: w.astype(jnp.bfloat16)
    embd_hi = bf(embd)
    embd_lo = bf(embd - embd_hi.astype(jnp.float32))
    col = lambda b: b.reshape(-1, 1)
    weights = (bf(W1.T), col(b1), bf(W2.T), col(b2), bf(W3.T), col(b3),
               bf(W4.T), col(b4), embd, embd_hi, embd_lo, bf(embd.T),
               bf(W5.T), col(b5), bf(W6.T), col(b6), bf(W7.T), col(b7),
               bf(W8.T), col(b8))

    out = pl.pallas_call(
        _body,
        grid=(nblk,),
        in_specs=[colT] + [full(w) for w in weights],
        out_specs=[colT, row2(D), row2(D),
                   pl.BlockSpec((1, K), lambda i: (0, 0))],
        out_shape=[
            jax.ShapeDtypeStruct((784, B), jnp.float32),
            jax.ShapeDtypeStruct((B, D), jnp.float32),
            jax.ShapeDtypeStruct((B, D), jnp.float32),
            jax.ShapeDtypeStruct((1, K), jnp.int32),
        ],
        scratch_shapes=[pltpu.VMEM((1, K), jnp.float32)],
        compiler_params=pltpu.CompilerParams(
            dimension_semantics=("arbitrary",)),
    )(XT, *weights)

    XRT, Z_enc, Z_dec, Zfi = out
    Zfe = _sc_gather(Z_enc, Zfi, K, D)
    X_recon = XRT.reshape(28, 28, B).transpose(2, 0, 1)[:, None, :, :]
    return (X_recon, Z_enc, Z_dec, Zfe)


# speedup vs baseline: 1.9814x; 1.0252x over previous
import jax
import jax.numpy as jnp
from jax.experimental import pallas as pl
from jax.experimental.pallas import tpu as pltpu
from jax.experimental.pallas import tpu_sc as plsc


def _dotbf(a, b):
    return jax.lax.dot_general(
        a.astype(jnp.bfloat16), b.astype(jnp.bfloat16), (((1,), (0,)), ((), ())),
        preferred_element_type=jnp.float32)


def _leaky(x):
    return jnp.where(x >= 0, x, 0.1 * x)


def _sc_gather(table, idx, n, d):
    @pl.kernel(out_type=jax.ShapeDtypeStruct((n, d), table.dtype),
               mesh=plsc.VectorSubcoreMesh(
                   core_axis_name="core", subcore_axis_name="subcore"))
    def k(tab_hbm, i_hbm, o_hbm):
        def body(i_vmem, o_vmem):
            pltpu.sync_copy(tab_hbm.at[i_vmem.at[0]], o_vmem)

        pltpu.emit_pipeline(
            body,
            grid=(n // 128,),
            in_specs=[pl.BlockSpec((1, 128), lambda i: (0, i))],
            out_specs=[pl.BlockSpec((128, d), lambda i: (i, 0))],
            core_axis_name=("core", "subcore"),
            dimension_semantics=(pltpu.PARALLEL,),
        )(i_hbm, o_hbm)

    return k(table, idx)


def _body(xt_ref, w1t, b1c, w2t, b2c, w3t, b3c, w4t, b4c,
          embd, embd_hi, embd_lo, embd_t,
          w5t, b5c, w6t, b6c, w7t, b7c, w8t, b8c,
          xrt_ref, zenc_ref, zdec_ref, zfi_ref, runmin_ref):
    i = pl.program_id(0)
    blk = xt_ref.shape[1]
    K, D = embd.shape

    ht = jnp.maximum(_dotbf(w1t[...], xt_ref[...]) + b1c[...], 0.0)
    ht = jnp.maximum(_dotbf(w2t[...], ht) + b2c[...], 0.0)
    ht = jnp.maximum(_dotbf(w3t[...], ht) + b3c[...], 0.0)
    zt = _dotbf(w4t[...], ht) + b4c[...]
    z = zt.T
    zenc_ref[...] = z

    qsq = jnp.sum(z * z, axis=1, keepdims=True)
    tsq = jnp.sum(embd[...] * embd[...], axis=1)
    g = _dotbf(z, embd_t[...])
    d2 = jnp.maximum(qsq + tsq[None, :] - 2.0 * g, 0.0)

    iota_k = jax.lax.broadcasted_iota(jnp.int32, (blk, K), 1)
    dmin = jnp.min(d2, axis=1, keepdims=True)
    idx = jnp.min(jnp.where(d2 == dmin, iota_k, K), axis=1, keepdims=True)
    onehot = (iota_k == idx).astype(jnp.bfloat16)
    zq = _dotbf(onehot, embd_hi[...]) + _dotbf(onehot, embd_lo[...])
    zdec_ref[...] = zq

    iota_r = jax.lax.broadcasted_iota(jnp.int32, (blk, K), 0)
    bmin = jnp.min(d2, axis=0).reshape(1, K)
    brow = jnp.min(jnp.where(d2 == bmin, iota_r, blk), axis=0).reshape(1, K)

    @pl.when(i == 0)
    def _():
        runmin_ref[...] = jnp.full(runmin_ref.shape, jnp.inf, jnp.float32)

    better = bmin < runmin_ref[...]
    runmin_ref[...] = jnp.where(better, bmin, runmin_ref[...])
    zfi_ref[...] = jnp.where(better, i * blk + brow, zfi_ref[...])

    dt = _leaky(_dotbf(w5t[...], zq.T) + b5c[...])
    dt = _leaky(_dotbf(w6t[...], dt) + b6c[...])
    dt = _leaky(_dotbf(w7t[...], dt) + b7c[...])
    xrt_ref[...] = jnp.tanh(_dotbf(w8t[...], dt) + b8c[...])


def kernel(X, W1, b1, W2, b2, W3, b3, W4, b4, embd, W5, b5, W6, b6, W7, b7, W8, b8):
    B = X.shape[0]
    K, D = embd.shape
    BLK = 1024
    nblk = B // BLK

    XT = X[:, 0].transpose(1, 2, 0).reshape(784, B)

    def full(arr):
        return pl.BlockSpec(arr.shape, lambda i: (0,) * arr.ndim)

    row2 = lambda n: pl.BlockSpec((BLK, n), lambda i: (i, 0))
    colT = pl.BlockSpec((784, BLK), lambda i: (0, i))
    bf = lambda w: w.astype(jnp.bfloat16)
    embd_hi = bf(embd)
    embd_lo = bf(embd - embd_hi.astype(jnp.float32))
    col = lambda b: b.reshape(-1, 1)
    weights = (bf(W1.T), col(b1), bf(W2.T), col(b2), bf(W3.T), col(b3),
               bf(W4.T), col(b4), embd, embd_hi, embd_lo, bf(embd.T),
               bf(W5.T), col(b5), bf(W6.T), col(b6), bf(W7.T), col(b7),
               bf(W8.T), col(b8))

    out = pl.pallas_call(
        _body,
        grid=(nblk,),
        in_specs=[colT] + [full(w) for w in weights],
        out_specs=[colT, row2(D), row2(D),
                   pl.BlockSpec((1, K), lambda i: (0, 0))],
        out_shape=[
            jax.ShapeDtypeStruct((784, B), jnp.float32),
            jax.ShapeDtypeStruct((B, D), jnp.float32),
            jax.ShapeDtypeStruct((B, D), jnp.float32),
            jax.ShapeDtypeStruct((1, K), jnp.int32),
        ],
        scratch_shapes=[pltpu.VMEM((1, K), jnp.float32)],
        compiler_params=pltpu.CompilerParams(
            dimension_semantics=("arbitrary",)),
    )(XT, *weights)

    XRT, Z_enc, Z_dec, Zfi = out
    Zfe = _sc_gather(Z_enc, Zfi, K, D)
    X_recon = XRT.reshape(28, 28, B).transpose(2, 0, 1)[:, None, :, :]
    return (X_recon, Z_enc, Z_dec, Zfe)


# speedup vs baseline: 1.9907x; 1.0047x over previous
import jax
import jax.numpy as jnp
from jax.experimental import pallas as pl
from jax.experimental.pallas import tpu as pltpu
from jax.experimental.pallas import tpu_sc as plsc


def _dotbf(a, b):
    return jax.lax.dot_general(
        a.astype(jnp.bfloat16), b.astype(jnp.bfloat16), (((1,), (0,)), ((), ())),
        preferred_element_type=jnp.float32)


def _leaky(x):
    return jnp.where(x >= 0, x, 0.1 * x)


def _sc_gather(table, idx, n, d):
    @pl.kernel(out_type=jax.ShapeDtypeStruct((n, d), table.dtype),
               mesh=plsc.VectorSubcoreMesh(
                   core_axis_name="core", subcore_axis_name="subcore"))
    def k(tab_hbm, i_hbm, o_hbm):
        def body(i_vmem, o_vmem):
            pltpu.sync_copy(tab_hbm.at[i_vmem.at[0]], o_vmem)

        pltpu.emit_pipeline(
            body,
            grid=(n // 128,),
            in_specs=[pl.BlockSpec((1, 128), lambda i: (0, i))],
            out_specs=[pl.BlockSpec((128, d), lambda i: (i, 0))],
            core_axis_name=("core", "subcore"),
            dimension_semantics=(pltpu.PARALLEL,),
        )(i_hbm, o_hbm)

    return k(table, idx)


def _body(xt_ref, w1t, b1c, w2t, b2c, w3t, b3c, w4t, b4c,
          embd, embd_hi, embd_lo, embd_t,
          w5t, b5c, w6t, b6c, w7t, b7c, w8t, b8c,
          xrt_ref, zenc_ref, zdec_ref, zfi_ref, runmin_ref):
    i = pl.program_id(0)
    blk = xt_ref.shape[1]
    K, D = embd.shape

    ht = jnp.maximum(_dotbf(w1t[...], xt_ref[...]) + b1c[...], 0.0)
    ht = jnp.maximum(_dotbf(w2t[...], ht) + b2c[...], 0.0)
    ht = jnp.maximum(_dotbf(w3t[...], ht) + b3c[...], 0.0)
    zt = _dotbf(w4t[...], ht) + b4c[...]
    z = zt.T
    zenc_ref[...] = z

    qsq = jnp.sum(z * z, axis=1, keepdims=True)
    tsq = jnp.sum(embd[...] * embd[...], axis=1)
    g = _dotbf(z, embd_t[...])
    d2 = jnp.maximum(qsq + tsq[None, :] - 2.0 * g, 0.0)

    iota_k = jax.lax.broadcasted_iota(jnp.int32, (blk, K), 1)
    dmin = jnp.min(d2, axis=1, keepdims=True)
    idx = jnp.min(jnp.where(d2 == dmin, iota_k, K), axis=1, keepdims=True)
    onehot = (iota_k == idx).astype(jnp.bfloat16)
    zq = _dotbf(onehot, embd_hi[...]) + _dotbf(onehot, embd_lo[...])
    zdec_ref[...] = zq

    iota_r = jax.lax.broadcasted_iota(jnp.int32, (blk, K), 0)
    bmin = jnp.min(d2, axis=0).reshape(1, K)
    brow = jnp.min(jnp.where(d2 == bmin, iota_r, blk), axis=0).reshape(1, K)

    @pl.when(i == 0)
    def _():
        runmin_ref[...] = jnp.full(runmin_ref.shape, jnp.inf, jnp.float32)

    better = bmin < runmin_ref[...]
    runmin_ref[...] = jnp.where(better, bmin, runmin_ref[...])
    zfi_ref[...] = jnp.where(better, i * blk + brow, zfi_ref[...])

    dt = _leaky(_dotbf(w5t[...], zq.T) + b5c[...])
    dt = _leaky(_dotbf(w6t[...], dt) + b6c[...])
    dt = _leaky(_dotbf(w7t[...], dt) + b7c[...])
    xrt_ref[...] = jnp.tanh(_dotbf(w8t[...], dt) + b8c[...])


def kernel(X, W1, b1, W2, b2, W3, b3, W4, b4, embd, W5, b5, W6, b6, W7, b7, W8, b8):
    B = X.shape[0]
    K, D = embd.shape
    BLK = 2048
    nblk = B // BLK

    XT = X[:, 0].transpose(1, 2, 0).reshape(784, B)

    def full(arr):
        return pl.BlockSpec(arr.shape, lambda i: (0,) * arr.ndim)

    row2 = lambda n: pl.BlockSpec((BLK, n), lambda i: (i, 0))
    colT = pl.BlockSpec((784, BLK), lambda i: (0, i))
    bf = lambda w: w.astype(jnp.bfloat16)
    embd_hi = bf(embd)
    embd_lo = bf(embd - embd_hi.astype(jnp.float32))
    col = lambda b: b.reshape(-1, 1)
    weights = (bf(W1.T), col(b1), bf(W2.T), col(b2), bf(W3.T), col(b3),
               bf(W4.T), col(b4), embd, embd_hi, embd_lo, bf(embd.T),
               bf(W5.T), col(b5), bf(W6.T), col(b6), bf(W7.T), col(b7),
               bf(W8.T), col(b8))

    out = pl.pallas_call(
        _body,
        grid=(nblk,),
        in_specs=[colT] + [full(w) for w in weights],
        out_specs=[colT, row2(D), row2(D),
                   pl.BlockSpec((1, K), lambda i: (0, 0))],
        out_shape=[
            jax.ShapeDtypeStruct((784, B), jnp.float32),
            jax.ShapeDtypeStruct((B, D), jnp.float32),
            jax.ShapeDtypeStruct((B, D), jnp.float32),
            jax.ShapeDtypeStruct((1, K), jnp.int32),
        ],
        scratch_shapes=[pltpu.VMEM((1, K), jnp.float32)],
        compiler_params=pltpu.CompilerParams(
            dimension_semantics=("arbitrary",)),
    )(XT, *weights)

    XRT, Z_enc, Z_dec, Zfi = out
    Zfe = _sc_gather(Z_enc, Zfi, K, D)
    X_recon = XRT.reshape(28, 28, B).transpose(2, 0, 1)[:, None, :, :]
    return (X_recon, Z_enc, Z_dec, Zfe)
